# baseline (speedup 1.0000x reference)
import functools

import jax
import jax.numpy as jnp
from jax import lax
from jax.experimental import pallas as pl
from jax.experimental.pallas import tpu as pltpu
from jax.experimental.pallas import tpu_sc as plsc

ALPHA = 0.1
K = 10
ROWB = 512

F32 = jnp.float32


def _mlp_body(x_ref, w1_ref, b1_ref, w2_ref, b2_ref, out_ref):
    xb = x_ref[...]
    h = jnp.maximum(
        jnp.dot(xb, w1_ref[...], preferred_element_type=F32) + b1_ref[...], 0.0)
    h2 = jnp.dot(h, w2_ref[...], preferred_element_type=F32) + b2_ref[...]
    pad = jnp.zeros((ROWB, 64 - h2.shape[1]), F32)
    full = jnp.concatenate([h2, pad], axis=1)
    out_ref[0] = full[:, :32]
    out_ref[1] = full[:, 32:]


def _mlp(x, W1, b1, W2, b2, Np):
    nblk = Np // ROWB
    feat = x.shape[1]
    hid = W1.shape[1]
    cls = W2.shape[1]
    return pl.pallas_call(
        _mlp_body,
        grid=(nblk,),
        in_specs=[
            pl.BlockSpec((ROWB, feat), lambda i: (i, 0)),
            pl.BlockSpec((feat, hid), lambda i: (0, 0)),
            pl.BlockSpec((1, hid), lambda i: (0, 0)),
            pl.BlockSpec((hid, cls), lambda i: (0, 0)),
            pl.BlockSpec((1, cls), lambda i: (0, 0)),
        ],
        out_specs=pl.BlockSpec((2, ROWB, 32), lambda i: (0, i, 0)),
        out_shape=jax.ShapeDtypeStruct((2, Np, 32), F32),
    )(x, W1, b1.reshape(1, hid), W2, b2.reshape(1, cls))


def _dinv_body(part_ref, h0_ref, dinv_ref, dinv2_ref, g0_ref):
    d = jnp.sum(part_ref[...], axis=0) + 1.0
    dinv = lax.rsqrt(d)
    dinv_ref[...] = dinv
    dinv2_ref[...] = 1.0 / d
    g0_ref[...] = h0_ref[...] * dinv[None, :, :]


def _dinv(deg_part, h0, Np):
    nblk = Np // ROWB
    return pl.pallas_call(
        _dinv_body,
        grid=(nblk,),
        in_specs=[pl.BlockSpec((32, ROWB, 1), lambda i: (0, i, 0)),
                  pl.BlockSpec((2, ROWB, 32), lambda i: (0, i, 0))],
        out_specs=[pl.BlockSpec((ROWB, 1), lambda i: (i, 0)),
                   pl.BlockSpec((ROWB, 1), lambda i: (i, 0)),
                   pl.BlockSpec((2, ROWB, 32), lambda i: (0, i, 0))],
        out_shape=[jax.ShapeDtypeStruct((Np, 1), F32),
                   jax.ShapeDtypeStruct((Np, 1), F32),
                   jax.ShapeDtypeStruct((2, Np, 32), F32)],
    )(deg_part.reshape(32, Np, 1), h0)


def _fin_body(g_ref, dinv_ref, out_ref, *, cls):
    h64 = jnp.concatenate([g_ref[0], g_ref[1]], axis=1) / dinv_ref[...]
    h = h64[:, :cls]
    m = jnp.max(h, axis=1, keepdims=True)
    e = jnp.exp(h - m)
    s = jnp.sum(e, axis=1, keepdims=True)
    out_ref[...] = h - m - jnp.log(s)


def _fin(g, dinv, N, Np, cls):
    nblk = Np // ROWB
    return pl.pallas_call(
        functools.partial(_fin_body, cls=cls),
        grid=(nblk,),
        in_specs=[pl.BlockSpec((2, ROWB, 32), lambda i: (0, i, 0)),
                  pl.BlockSpec((ROWB, 1), lambda i: (i, 0))],
        out_specs=pl.BlockSpec((ROWB, cls), lambda i: (i, 0)),
        out_shape=jax.ShapeDtypeStruct((N, cls), F32),
    )(g.reshape(2, Np, 32), dinv)


def _make_deg(Np, Ep):
    mesh = plsc.VectorSubcoreMesh(core_axis_name="c", subcore_axis_name="s")
    rows_all = Ep // 128
    rows_w = rows_all // 32
    n_chunks = rows_w // 8
    seg = Np // 16

    @functools.partial(
        pl.kernel,
        out_type=jax.ShapeDtypeStruct((32 * Np,), F32),
        mesh=mesh,
        scratch_types=[
            pltpu.VMEM((Np,), F32),
            pltpu.VMEM((8, 128), jnp.int32),
        ],
        compiler_params=pltpu.CompilerParams(needs_layout_passes=False, use_tc_tiling_on_sc=False),
    )
    def deg_kernel(dst_hbm, out_hbm, hist, dstbuf):
        c = lax.axis_index("c")
        sid = lax.axis_index("s")
        wid = c * 16 + sid
        ones = jnp.ones((16,), F32)

        def zero_body(i, _):
            hist[pl.ds(i * 16, 16)] = jnp.zeros((16,), F32)
            return 0
        lax.fori_loop(0, Np // 16, zero_body, 0)

        def chunk_body(j, _):
            pltpu.sync_copy(dst_hbm.at[pl.ds(wid * rows_w + j * 8, 8), :],
                            dstbuf)
            for r in range(8):
                for q in range(8):
                    idx = dstbuf[r, pl.ds(q * 16, 16)]
                    plsc.addupdate_scatter(hist, [idx], ones)
            return 0
        lax.fori_loop(0, n_chunks, chunk_body, 0)

        pltpu.sync_copy(hist, out_hbm.at[pl.ds(wid * Np, Np)])

    return deg_kernel


def _make_prop(Np, Ep):
    mesh = plsc.VectorSubcoreMesh(core_axis_name="c", subcore_axis_name="s")
    rows_all = Ep // 128
    rows_t = rows_all // 16
    BR = 2
    NB = rows_t // BR
    seg = Np // 16
    RC = seg // 64
    NC = seg // RC
    beta = 1.0 - ALPHA

    @functools.partial(
        pl.kernel,
        out_type=jax.ShapeDtypeStruct((2 * Np, 32), F32),
        mesh=mesh,
        scratch_types=[
            [pltpu.VMEM((BR, 2, 128), jnp.int32) for _ in range(3)],
            [pltpu.VMEM((BR * 128, 32), F32) for _ in range(2)],
            [pltpu.VMEM((RC, 32), F32) for _ in range(2)],
            [pltpu.VMEM((RC, 32), F32) for _ in range(2)],
            pltpu.VMEM((seg,), F32),
            pltpu.VMEM_SHARED((Np, 32), F32),
            pltpu.SemaphoreType.DMA,
            pltpu.SemaphoreType.DMA,
            pltpu.SemaphoreType.DMA,
            pltpu.SemaphoreType.DMA,
            pltpu.SemaphoreType.DMA,
        ],
        compiler_params=pltpu.CompilerParams(needs_layout_passes=False, use_tc_tiling_on_sc=False),
    )
    def prop_kernel(g0_hbm, eidx_hbm, dinv2_hbm, g_hbm,
                    idxI, rows, aggbufs, g0bufs, d2buf, agg,
                    isem, gsem, ssem, csem, wsem):
        c = lax.axis_index("c")
        sid = lax.axis_index("s")
        nbase = sid * seg
        ebase = sid * rows_t
        coff = c * Np

        def idx_fire(b, ibuf):
            er = pl.ds(ebase + b * BR, BR)
            pltpu.async_copy(eidx_hbm.at[c, er, :, :], ibuf, isem)

        def idx_wait(b, ibuf):
            er = pl.ds(ebase + b * BR, BR)
            pltpu.make_async_copy(eidx_hbm.at[c, er, :, :], ibuf,
                                  isem).wait()

        def gather_fire(ibuf, rbuf):
            for r in range(BR):
                pltpu.async_copy(g_hbm.at[ibuf.at[r, 0]],
                                 rbuf.at[pl.ds(r * 128, 128)], gsem)

        def gather_wait(rbuf):
            pltpu.make_async_copy(g_hbm.at[pl.ds(0, BR * 128)], rbuf,
                                  gsem).wait()

        def scatter_fire(ibuf, rbuf):
            for r in range(BR):
                pltpu.async_copy(rbuf.at[pl.ds(r * 128, 128)],
                                 agg.at[ibuf.at[r, 1]], ssem, add=True)

        def scatter_drain(rbuf):
            pltpu.make_async_copy(rbuf, agg.at[pl.ds(0, BR * 128)],
                                  ssem).wait()

        pltpu.sync_copy(dinv2_hbm.at[pl.ds(nbase, seg)], d2buf)

        def init_body(cc, _):
            loc0 = nbase + cc * RC
            rs = pl.ds(coff + loc0, RC)
            pltpu.sync_copy(g0_hbm.at[rs], aggbufs[0])
            pltpu.sync_copy(aggbufs[0], g_hbm.at[rs])
            pltpu.sync_copy(aggbufs[0], agg.at[pl.ds(loc0, RC)])
            return 0
        lax.fori_loop(0, NC, init_body, 0)
        plsc.subcore_barrier()

        def iter_body(t, _):
            with jax.named_scope("phaseB"):
                idx_fire(0, idxI[0])
                idx_wait(0, idxI[0])
                gather_fire(idxI[0], rows[0])
                idx_fire(1, idxI[1])

            def step(b, rcur, rnxt, icur, inxt, innxt):
                @pl.when(b + 1 < NB)
                def _():
                    idx_wait(b + 1, inxt)

                @pl.when(b >= 1)
                def _():
                    scatter_drain(rnxt)

                @pl.when(b + 1 < NB)
                def _():
                    gather_fire(inxt, rnxt)

                @pl.when(b + 2 < NB)
                def _():
                    idx_fire(b + 2, innxt)

                gather_wait(rcur)
                scatter_fire(icur, rcur)

            def pb_body(b, _):
                m = b % 6
                for k in range(6):
                    @pl.when(m == k)
                    def _(k=k):
                        step(b, rows[k % 2], rows[(k + 1) % 2],
                             idxI[k % 3], idxI[(k + 1) % 3],
                             idxI[(k + 2) % 3])
                return 0
            with jax.named_scope("phaseB2"):
                lax.fori_loop(0, NB, pb_body, 0)
                scatter_drain(rows[(NB - 1) % 2])
                plsc.subcore_barrier()

            def comb_loads(cc, gb):
                loc0 = nbase + cc * RC
                pltpu.async_copy(g0_hbm.at[pl.ds(coff + loc0, RC)], gb, csem)

            def comb_loads_wait(gb):
                pltpu.make_async_copy(g0_hbm.at[pl.ds(0, RC)], gb,
                                      csem).wait()

            def comb_wb(cc, ab):
                loc0 = nbase + cc * RC
                pltpu.async_copy(ab, g_hbm.at[pl.ds(coff + loc0, RC)], wsem)

            def comb_wb_drain(ab):
                pltpu.make_async_copy(ab, g_hbm.at[pl.ds(0, RC)], wsem).wait()

            def comb_step(cc, ab, gb, abo, gbo):
                @pl.when(cc >= 1)
                def _():
                    comb_wb_drain(abo)

                @pl.when(cc + 1 < NC)
                def _():
                    comb_loads(cc + 1, gbo)
                loc0 = nbase + cc * RC
                pltpu.sync_copy(agg.at[pl.ds(loc0, RC)], ab)
                comb_loads_wait(gb)

                def row_body(i, _):
                    idxv = jnp.zeros((16,), jnp.int32) + (cc * RC + i)
                    d2v = plsc.load_gather(d2buf, [idxv]) * beta
                    for v in range(2):
                        s = pl.ds(v * 16, 16)
                        ab[i, s] = d2v * ab[i, s] + ALPHA * gb[i, s]
                    return 0
                lax.fori_loop(0, RC, row_body, 0)
                pltpu.sync_copy(ab, agg.at[pl.ds(loc0, RC)])
                comb_wb(cc, ab)

            with jax.named_scope("phaseC"):
                comb_loads(0, g0bufs[0])

                def comb_body(cc, _):
                    @pl.when(cc % 2 == 0)
                    def _():
                        comb_step(cc, aggbufs[0], g0bufs[0],
                                  aggbufs[1], g0bufs[1])

                    @pl.when(cc % 2 == 1)
                    def _():
                        comb_step(cc, aggbufs[1], g0bufs[1],
                                  aggbufs[0], g0bufs[0])
                    return 0
                lax.fori_loop(0, NC, comb_body, 0)
                comb_wb_drain(aggbufs[(NC - 1) % 2])
                plsc.subcore_barrier()
            return 0

        lax.fori_loop(0, K, iter_body, 0)

    return prop_kernel


def kernel(x, edge_index, W1, b1, W2, b2):
    N, feat = x.shape
    cls = W2.shape[1]
    E = edge_index.shape[1]

    Np = ((N + ROWB - 1) // ROWB) * ROWB
    while Np % (16 * 16) != 0:
        Np += ROWB
    Ep = ((E + 32 * 1024 - 1) // (32 * 1024)) * (32 * 1024)

    src = edge_index[0]
    dst = edge_index[1]
    if Ep != E:
        fill = jnp.full((Ep - E,), N, jnp.int32)
        src = jnp.concatenate([src, fill])
        dst = jnp.concatenate([dst, fill])
    src_adj = jnp.stack([src, src + Np]).reshape(2, Ep // 128, 128)
    dst3 = dst.reshape(Ep // 128, 128)
    eidx = jnp.concatenate(
        [src_adj[:, :, None, :],
         jnp.broadcast_to(dst3[None, :, None, :], (2, Ep // 128, 1, 128))],
        axis=2)

    deg_part = _make_deg(Np, Ep)(dst3)
    h0 = _mlp(x, W1, b1, W2, b2, Np)
    dinv, dinv2, g0 = _dinv(deg_part, h0, Np)

    gK = _make_prop(Np, Ep)(g0.reshape(2 * Np, 32), eidx, dinv2.reshape(Np))
    return _fin(gK, dinv, N, Np, cls)


# speedup vs baseline: 1.0738x; 1.0738x over previous
import functools

import jax
import jax.numpy as jnp
from jax import lax
from jax.experimental import pallas as pl
from jax.experimental.pallas import tpu as pltpu
from jax.experimental.pallas import tpu_sc as plsc

ALPHA = 0.1
K = 10
ROWB = 512

F32 = jnp.float32


def _mlp_body(x_ref, w1_ref, b1_ref, w2_ref, b2_ref, dinv_ref, out_ref):
    xb = x_ref[...]
    h = jnp.maximum(
        jnp.dot(xb, w1_ref[...], preferred_element_type=F32) + b1_ref[...], 0.0)
    h2 = jnp.dot(h, w2_ref[...], preferred_element_type=F32) + b2_ref[...]
    scaled = h2 * (dinv_ref[...] * ALPHA)
    pad = jnp.zeros((ROWB, 64 - scaled.shape[1]), F32)
    full = jnp.concatenate([scaled, pad], axis=1)
    out_ref[0] = full[:, :32]
    out_ref[1] = full[:, 32:]


def _mlp(x, W1, b1, W2, b2, dinv, Np):
    nblk = Np // ROWB
    feat = x.shape[1]
    hid = W1.shape[1]
    cls = W2.shape[1]
    return pl.pallas_call(
        _mlp_body,
        grid=(nblk,),
        in_specs=[
            pl.BlockSpec((ROWB, feat), lambda i: (i, 0)),
            pl.BlockSpec((feat, hid), lambda i: (0, 0)),
            pl.BlockSpec((1, hid), lambda i: (0, 0)),
            pl.BlockSpec((hid, cls), lambda i: (0, 0)),
            pl.BlockSpec((1, cls), lambda i: (0, 0)),
            pl.BlockSpec((ROWB, 1), lambda i: (i, 0)),
        ],
        out_specs=pl.BlockSpec((2, ROWB, 32), lambda i: (0, i, 0)),
        out_shape=jax.ShapeDtypeStruct((2, Np, 32), F32),
    )(x, W1, b1.reshape(1, hid), W2, b2.reshape(1, cls), dinv)


def _dinv_body(part_ref, dinv_ref, dinv2_ref):
    d = jnp.sum(part_ref[...], axis=0) + 1.0
    dinv_ref[...] = lax.rsqrt(d)
    dinv2_ref[...] = (1.0 - ALPHA) / d


def _dinv(deg_part, Np):
    nblk = Np // ROWB
    return pl.pallas_call(
        _dinv_body,
        grid=(nblk,),
        in_specs=[pl.BlockSpec((32, ROWB, 1), lambda i: (0, i, 0))],
        out_specs=[pl.BlockSpec((ROWB, 1), lambda i: (i, 0)),
                   pl.BlockSpec((ROWB, 1), lambda i: (i, 0))],
        out_shape=[jax.ShapeDtypeStruct((Np, 1), F32),
                   jax.ShapeDtypeStruct((Np, 1), F32)],
    )(deg_part.reshape(32, Np, 1))


def _fin_body(g_ref, dinv_ref, out_ref, *, cls):
    h64 = jnp.concatenate([g_ref[0], g_ref[1]], axis=1) / dinv_ref[...]
    h = h64[:, :cls]
    m = jnp.max(h, axis=1, keepdims=True)
    e = jnp.exp(h - m)
    s = jnp.sum(e, axis=1, keepdims=True)
    out_ref[...] = h - m - jnp.log(s)


def _fin(g, dinv, N, Np, cls):
    nblk = Np // ROWB
    return pl.pallas_call(
        functools.partial(_fin_body, cls=cls),
        grid=(nblk,),
        in_specs=[pl.BlockSpec((2, ROWB, 32), lambda i: (0, i, 0)),
                  pl.BlockSpec((ROWB, 1), lambda i: (i, 0))],
        out_specs=pl.BlockSpec((ROWB, cls), lambda i: (i, 0)),
        out_shape=jax.ShapeDtypeStruct((N, cls), F32),
    )(g.reshape(2, Np, 32), dinv)


def _make_deg(Np, Ep):
    mesh = plsc.VectorSubcoreMesh(core_axis_name="c", subcore_axis_name="s")
    rows_all = Ep // 128
    rows_w = rows_all // 32
    n_chunks = rows_w // 8
    seg = Np // 16

    @functools.partial(
        pl.kernel,
        out_type=jax.ShapeDtypeStruct((32 * Np,), F32),
        mesh=mesh,
        scratch_types=[
            pltpu.VMEM((Np,), F32),
            pltpu.VMEM((8, 128), jnp.int32),
        ],
        compiler_params=pltpu.CompilerParams(needs_layout_passes=False, use_tc_tiling_on_sc=False),
    )
    def deg_kernel(dst_hbm, out_hbm, hist, dstbuf):
        c = lax.axis_index("c")
        sid = lax.axis_index("s")
        wid = c * 16 + sid
        ones = jnp.ones((16,), F32)

        def zero_body(i, _):
            hist[pl.ds(i * 16, 16)] = jnp.zeros((16,), F32)
            return 0
        lax.fori_loop(0, Np // 16, zero_body, 0)

        def chunk_body(j, _):
            pltpu.sync_copy(dst_hbm.at[pl.ds(wid * rows_w + j * 8, 8), :],
                            dstbuf)
            for r in range(8):
                for q in range(8):
                    idx = dstbuf[r, pl.ds(q * 16, 16)]
                    plsc.addupdate_scatter(hist, [idx], ones)
            return 0
        lax.fori_loop(0, n_chunks, chunk_body, 0)

        pltpu.sync_copy(hist, out_hbm.at[pl.ds(wid * Np, Np)])

    return deg_kernel


def _make_prop(Np, Ep):
    mesh = plsc.VectorSubcoreMesh(core_axis_name="c", subcore_axis_name="s")
    rows_all = Ep // 128
    rows_t = rows_all // 16
    BR = 2
    NB = rows_t // BR
    seg = Np // 16
    RC = seg // 64
    NC = seg // RC
    beta = 1.0 - ALPHA

    @functools.partial(
        pl.kernel,
        out_type=jax.ShapeDtypeStruct((2 * Np, 32), F32),
        mesh=mesh,
        scratch_types=[
            [pltpu.VMEM((BR, 2, 128), jnp.int32) for _ in range(3)],
            [pltpu.VMEM((BR * 128, 32), F32) for _ in range(2)],
            [pltpu.VMEM((RC, 32), F32) for _ in range(2)],
            [pltpu.VMEM((RC, 32), F32) for _ in range(2)],
            pltpu.VMEM((seg,), F32),
            pltpu.VMEM_SHARED((Np, 32), F32),
            pltpu.SemaphoreType.DMA,
            pltpu.SemaphoreType.DMA,
            pltpu.SemaphoreType.DMA,
            pltpu.SemaphoreType.DMA,
            pltpu.SemaphoreType.DMA,
        ],
        compiler_params=pltpu.CompilerParams(needs_layout_passes=False, use_tc_tiling_on_sc=False),
    )
    def prop_kernel(g0_hbm, eidx_hbm, dinv2_hbm, g_hbm,
                    idxI, rows, aggbufs, g0bufs, d2buf, agg,
                    isem, gsem, ssem, csem, wsem):
        c = lax.axis_index("c")
        sid = lax.axis_index("s")
        nbase = sid * seg
        ebase = sid * rows_t
        coff = c * Np

        def idx_fire(b, ibuf):
            er = pl.ds(ebase + b * BR, BR)
            pltpu.async_copy(eidx_hbm.at[c, er, :, :], ibuf, isem)

        def idx_wait(b, ibuf):
            er = pl.ds(ebase + b * BR, BR)
            pltpu.make_async_copy(eidx_hbm.at[c, er, :, :], ibuf,
                                  isem).wait()

        def gather_fire(ibuf, rbuf):
            for r in range(BR):
                pltpu.async_copy(g_hbm.at[ibuf.at[r, 0]],
                                 rbuf.at[pl.ds(r * 128, 128)], gsem)

        def gather_wait(rbuf):
            pltpu.make_async_copy(g_hbm.at[pl.ds(0, BR * 128)], rbuf,
                                  gsem).wait()

        def scatter_fire(ibuf, rbuf):
            for r in range(BR):
                pltpu.async_copy(rbuf.at[pl.ds(r * 128, 128)],
                                 agg.at[ibuf.at[r, 1]], ssem, add=True)

        def scatter_drain(rbuf):
            pltpu.make_async_copy(rbuf, agg.at[pl.ds(0, BR * 128)],
                                  ssem).wait()

        pltpu.sync_copy(dinv2_hbm.at[pl.ds(nbase, seg)], d2buf)

        def init_body(cc, _):
            loc0 = nbase + cc * RC
            rs = pl.ds(coff + loc0, RC)
            pltpu.sync_copy(g0_hbm.at[rs], aggbufs[0])

            def unscale(i, _):
                for v in range(2):
                    s = pl.ds(v * 16, 16)
                    aggbufs[0][i, s] = aggbufs[0][i, s] * (1.0 / ALPHA)
                return 0
            lax.fori_loop(0, RC, unscale, 0)
            pltpu.sync_copy(aggbufs[0], g_hbm.at[rs])
            pltpu.sync_copy(aggbufs[0], agg.at[pl.ds(loc0, RC)])
            return 0
        lax.fori_loop(0, NC, init_body, 0)
        plsc.subcore_barrier()

        def iter_body(t, _):
            with jax.named_scope("phaseB"):
                idx_fire(0, idxI[0])
                idx_wait(0, idxI[0])
                gather_fire(idxI[0], rows[0])
                idx_fire(1, idxI[1])

            def step(b, rcur, rnxt, icur, inxt, innxt):
                @pl.when(b + 1 < NB)
                def _():
                    idx_wait(b + 1, inxt)

                @pl.when(b >= 1)
                def _():
                    scatter_drain(rnxt)

                @pl.when(b + 1 < NB)
                def _():
                    gather_fire(inxt, rnxt)

                @pl.when(b + 2 < NB)
                def _():
                    idx_fire(b + 2, innxt)

                gather_wait(rcur)
                scatter_fire(icur, rcur)

            def pb_body(b, _):
                m = b % 6
                for k in range(6):
                    @pl.when(m == k)
                    def _(k=k):
                        step(b, rows[k % 2], rows[(k + 1) % 2],
                             idxI[k % 3], idxI[(k + 1) % 3],
                             idxI[(k + 2) % 3])
                return 0
            with jax.named_scope("phaseB2"):
                lax.fori_loop(0, NB, pb_body, 0)
                scatter_drain(rows[(NB - 1) % 2])
                plsc.subcore_barrier()

            def comb_loads(cc, gb):
                loc0 = nbase + cc * RC
                pltpu.async_copy(g0_hbm.at[pl.ds(coff + loc0, RC)], gb, csem)

            def comb_loads_wait(gb):
                pltpu.make_async_copy(g0_hbm.at[pl.ds(0, RC)], gb,
                                      csem).wait()

            def comb_wb(cc, ab):
                loc0 = nbase + cc * RC
                pltpu.async_copy(ab, g_hbm.at[pl.ds(coff + loc0, RC)], wsem)

            def comb_wb_drain(ab):
                pltpu.make_async_copy(ab, g_hbm.at[pl.ds(0, RC)], wsem).wait()

            def comb_step(cc, ab, gb, abo, gbo):
                @pl.when(cc >= 1)
                def _():
                    comb_wb_drain(abo)

                @pl.when(cc + 1 < NC)
                def _():
                    comb_loads(cc + 1, gbo)
                loc0 = nbase + cc * RC
                pltpu.sync_copy(agg.at[pl.ds(loc0, RC)], ab)
                comb_loads_wait(gb)

                def row_body(i, _):
                    idxv = jnp.zeros((16,), jnp.int32) + (cc * RC + i)
                    d2v = plsc.load_gather(d2buf, [idxv])
                    for v in range(2):
                        s = pl.ds(v * 16, 16)
                        ab[i, s] = d2v * ab[i, s] + gb[i, s]
                    return 0
                lax.fori_loop(0, RC, row_body, 0)
                pltpu.sync_copy(ab, agg.at[pl.ds(loc0, RC)])
                comb_wb(cc, ab)

            with jax.named_scope("phaseC"):
                comb_loads(0, g0bufs[0])

                def comb_body(cc, _):
                    @pl.when(cc % 2 == 0)
                    def _():
                        comb_step(cc, aggbufs[0], g0bufs[0],
                                  aggbufs[1], g0bufs[1])

                    @pl.when(cc % 2 == 1)
                    def _():
                        comb_step(cc, aggbufs[1], g0bufs[1],
                                  aggbufs[0], g0bufs[0])
                    return 0
                lax.fori_loop(0, NC, comb_body, 0)
                comb_wb_drain(aggbufs[(NC - 1) % 2])
                plsc.subcore_barrier()
            return 0

        lax.fori_loop(0, K, iter_body, 0)

    return prop_kernel


def kernel(x, edge_index, W1, b1, W2, b2):
    N, feat = x.shape
    cls = W2.shape[1]
    E = edge_index.shape[1]

    Np = ((N + ROWB - 1) // ROWB) * ROWB
    while Np % (16 * 16) != 0:
        Np += ROWB
    Ep = ((E + 32 * 1024 - 1) // (32 * 1024)) * (32 * 1024)

    src = edge_index[0]
    dst = edge_index[1]
    if Ep != E:
        fill = jnp.full((Ep - E,), N, jnp.int32)
        src = jnp.concatenate([src, fill])
        dst = jnp.concatenate([dst, fill])
    src_adj = jnp.stack([src, src + Np]).reshape(2, Ep // 128, 128)
    dst3 = dst.reshape(Ep // 128, 128)
    eidx = jnp.concatenate(
        [src_adj[:, :, None, :],
         jnp.broadcast_to(dst3[None, :, None, :], (2, Ep // 128, 1, 128))],
        axis=2)

    deg_part = _make_deg(Np, Ep)(dst3)
    dinv, dinv2 = _dinv(deg_part, Np)

    g0 = _mlp(x, W1, b1, W2, b2, dinv, Np).reshape(2 * Np, 32)
    gK = _make_prop(Np, Ep)(g0, eidx, dinv2.reshape(Np))
    return _fin(gK, dinv, N, Np, cls)
